# baseline (speedup 1.0000x reference)
import functools

import jax
import jax.numpy as jnp
from jax import lax
from jax.experimental import pallas as pl
from jax.experimental.pallas import tpu as pltpu
from jax.experimental.pallas import tpu_sc as plsc

N = 10000
E = 320000
D = 128
DH = D // 2

NC = 2
NS = 16
NW = NC * NS

K = 128
C = 160
K2 = 64
C2 = 160
NB = 4
PER_W = C * K
EPAD = NS * PER_W

ROWS_PER_TILE = 632
ACC_ROWS = NS * ROWS_PER_TILE
DEG_W = 8
DEG_PER_TILE = 640
DEG_ROWS = NS * DEG_PER_TILE

_MESH = plsc.VectorSubcoreMesh(
    core_axis_name="c", subcore_axis_name="s", num_cores=NC, num_subcores=NS
)

_MSG_CHUNKS = tuple((i * 64, 64) for i in range(9)) + ((576, 56),)


@functools.partial(
    pl.kernel,
    out_type=jax.ShapeDtypeStruct((NC, DEG_ROWS, DEG_W), jnp.float32),
    mesh=_MESH,
    scratch_types=[
        pltpu.VMEM((C, K), jnp.int32),
        pltpu.VMEM((K, DEG_W), jnp.float32),
        pltpu.VMEM((DEG_PER_TILE, DEG_W), jnp.float32),
        pltpu.VMEM_SHARED((DEG_ROWS, DEG_W), jnp.float32),
    ],
    compiler_params=pltpu.CompilerParams(use_tc_tiling_on_sc=False),
)
def _deg_kernel(dst3_hbm, ones_hbm, zdeg_hbm, deg_out, idx_all, ones_v, zbuf, acc):
    c = lax.axis_index("c")
    s = lax.axis_index("s")
    pltpu.sync_copy(dst3_hbm.at[s], idx_all)
    pltpu.sync_copy(ones_hbm, ones_v)
    pltpu.sync_copy(zdeg_hbm, zbuf)
    row0 = s * DEG_PER_TILE
    pltpu.sync_copy(zbuf, acc.at[pl.ds(row0, DEG_PER_TILE)])
    plsc.subcore_barrier()

    half = C // 2

    def body(i, carry):
        cc = c * half + i
        pltpu.sync_copy(ones_v, acc.at[idx_all.at[cc]], add=True)
        return carry

    lax.fori_loop(0, half, body, 0)
    plsc.subcore_barrier()
    pltpu.sync_copy(acc.at[pl.ds(row0, DEG_PER_TILE)], zbuf)
    pltpu.sync_copy(zbuf, deg_out.at[c, pl.ds(row0, DEG_PER_TILE)])


@functools.partial(
    pl.kernel,
    out_type=jax.ShapeDtypeStruct((NC, ACC_ROWS, D), jnp.float32),
    mesh=_MESH,
    scratch_types=[
        pltpu.VMEM((NB, 2, K2), jnp.int32),
        [pltpu.VMEM((K2, D), jnp.float32) for _ in range(NB)],
        pltpu.VMEM((64, D), jnp.float32),
        pltpu.VMEM_SHARED((ACC_ROWS, D), jnp.float32),
        [pltpu.SemaphoreType.DMA for _ in range(NB)],
        [pltpu.SemaphoreType.DMA for _ in range(NB)],
    ],
    compiler_params=pltpu.CompilerParams(use_tc_tiling_on_sc=False),
)
def _msg_kernel(h2_hbm, sd4_hbm, zmsg_hbm, part_out,
                sdx, rows, zbuf, acc, sem_g, sem_s):
    c = lax.axis_index("c")
    s = lax.axis_index("s")
    wid = c * NS + s
    pltpu.sync_copy(zmsg_hbm, zbuf)
    base_row = s * ROWS_PER_TILE
    for off, cnt in _MSG_CHUNKS:
        pltpu.sync_copy(zbuf.at[pl.ds(0, cnt)], acc.at[pl.ds(base_row + off, cnt)])
    plsc.subcore_barrier()

    def body(tb, carry):
        for b in range(NB):
            @pl.when(tb > 0)
            def _():
                pltpu.make_async_copy(rows[b], acc.at[sdx.at[b, 1]], sem_s[b]).wait()
        pltpu.sync_copy(sd4_hbm.at[wid, pl.ds(tb * NB, NB)], sdx)
        for b in range(NB):
            pltpu.async_copy(h2_hbm.at[sdx.at[b, 0]], rows[b], sem_g[b])
        for b in range(NB):
            pltpu.make_async_copy(h2_hbm.at[sdx.at[b, 0]], rows[b], sem_g[b]).wait()
            pltpu.async_copy(rows[b], acc.at[sdx.at[b, 1]], sem_s[b], add=True)
        return carry

    lax.fori_loop(0, C2 // NB, body, 0)
    for b in range(NB):
        pltpu.make_async_copy(rows[b], acc.at[sdx.at[b, 1]], sem_s[b]).wait()
    plsc.subcore_barrier()
    for off, cnt in _MSG_CHUNKS:
        pltpu.sync_copy(acc.at[pl.ds(base_row + off, cnt)], zbuf.at[pl.ds(0, cnt)])
        pltpu.sync_copy(zbuf.at[pl.ds(0, cnt)], part_out.at[c, pl.ds(base_row + off, cnt)])


R = 1000


def _pre_body(x_ref, d0_ref, d1_ref, w_ref, wr_ref, b_ref, br_ref,
              h2_ref, base_ref, dis_ref):
    x = x_ref[...]
    deg = 1.0 + d0_ref[...][:, 0:1] + d1_ref[...][:, 0:1]
    dis = lax.rsqrt(deg)
    h2 = jnp.dot(x, w_ref[...], preferred_element_type=jnp.float32) * dis
    base = (h2 * dis
            + jnp.dot(x, wr_ref[...], preferred_element_type=jnp.float32)
            + b_ref[...] + br_ref[...])
    h2_ref[...] = h2
    base_ref[...] = base
    dis_ref[...] = dis


_pre_call = pl.pallas_call(
    _pre_body,
    grid=(N // R,),
    in_specs=[
        pl.BlockSpec((R, D), lambda i: (i, 0)),
        pl.BlockSpec((R, DEG_W), lambda i: (i, 0)),
        pl.BlockSpec((R, DEG_W), lambda i: (i, 0)),
        pl.BlockSpec((D, D), lambda i: (0, 0)),
        pl.BlockSpec((D, D), lambda i: (0, 0)),
        pl.BlockSpec((1, D), lambda i: (0, 0)),
        pl.BlockSpec((1, D), lambda i: (0, 0)),
    ],
    out_specs=[
        pl.BlockSpec((R, D), lambda i: (i, 0)),
        pl.BlockSpec((R, D), lambda i: (i, 0)),
        pl.BlockSpec((R, 1), lambda i: (i, 0)),
    ],
    out_shape=[
        jax.ShapeDtypeStruct((N, D), jnp.float32),
        jax.ShapeDtypeStruct((N, D), jnp.float32),
        jax.ShapeDtypeStruct((N, 1), jnp.float32),
    ],
)


def _post_body(p_ref, base_ref, dis_ref, o_ref):
    p = p_ref[...]
    o_ref[...] = jnp.maximum((p[0] + p[1]) * dis_ref[...] + base_ref[...], 0.0)


_post_call = pl.pallas_call(
    _post_body,
    grid=(N // R,),
    in_specs=[
        pl.BlockSpec((NC, R, D), lambda i: (0, i, 0)),
        pl.BlockSpec((R, D), lambda i: (i, 0)),
        pl.BlockSpec((R, 1), lambda i: (i, 0)),
    ],
    out_specs=pl.BlockSpec((R, D), lambda i: (i, 0)),
    out_shape=jax.ShapeDtypeStruct((N, D), jnp.float32),
)


def kernel(x, edge_index, W, b, W_res, b_res):
    ei = edge_index.astype(jnp.int32)
    npad = EPAD - E
    pad_src = jnp.zeros((npad,), jnp.int32)
    pad_dst = N + (jnp.arange(npad, dtype=jnp.int32) & 15)
    src_pad = jnp.concatenate([ei[0], pad_src])
    dst_pad = jnp.concatenate([ei[1], pad_dst])
    dst3 = dst_pad.reshape(NS, C, K)
    sd4 = jnp.stack(
        [src_pad.reshape(NW, C2, K2), dst_pad.reshape(NW, C2, K2)], axis=2
    )

    ones8 = jnp.zeros((K, DEG_W), jnp.float32).at[:, 0].set(1.0)
    zdeg = jnp.zeros((DEG_PER_TILE, DEG_W), jnp.float32)
    zmsg = jnp.zeros((64, D), jnp.float32)

    deg = _deg_kernel(dst3, ones8, zdeg)
    h2, base, dis = _pre_call(
        x, deg[0], deg[1], W, W_res, b.reshape(1, D), b_res.reshape(1, D)
    )
    part = _msg_kernel(h2, sd4, zmsg)
    return _post_call(part, base, dis)


# speedup vs baseline: 1.4174x; 1.4174x over previous
import functools

import jax
import jax.numpy as jnp
from jax import lax
from jax.experimental import pallas as pl
from jax.experimental.pallas import tpu as pltpu
from jax.experimental.pallas import tpu_sc as plsc

N = 10000
E = 320000
D = 128
DH = D // 2

NC = 2
NS = 16
NW = NC * NS

K = 128
C = 160
PER_W = C * K
EPAD = NS * PER_W

ROWS_PER_TILE = 632
ACC_ROWS = NS * ROWS_PER_TILE
DEG_W = 8
DEG_PER_TILE = 640
DEG_ROWS = NS * DEG_PER_TILE

_MESH = plsc.VectorSubcoreMesh(
    core_axis_name="c", subcore_axis_name="s", num_cores=NC, num_subcores=NS
)

_MSG_CHUNKS = ((0, 128), (128, 128), (256, 128), (384, 128), (512, 120))


@functools.partial(
    pl.kernel,
    out_type=jax.ShapeDtypeStruct((NC, DEG_ROWS, DEG_W), jnp.float32),
    mesh=_MESH,
    scratch_types=[
        pltpu.VMEM((C, K), jnp.int32),
        pltpu.VMEM((K, DEG_W), jnp.float32),
        pltpu.VMEM((DEG_PER_TILE, DEG_W), jnp.float32),
        pltpu.VMEM_SHARED((DEG_ROWS, DEG_W), jnp.float32),
    ],
    compiler_params=pltpu.CompilerParams(use_tc_tiling_on_sc=False),
)
def _deg_kernel(dst3_hbm, ones_hbm, zdeg_hbm, deg_out, idx_all, ones_v, zbuf, acc):
    c = lax.axis_index("c")
    s = lax.axis_index("s")
    pltpu.sync_copy(dst3_hbm.at[s], idx_all)
    pltpu.sync_copy(ones_hbm, ones_v)
    pltpu.sync_copy(zdeg_hbm, zbuf)
    row0 = s * DEG_PER_TILE
    pltpu.sync_copy(zbuf, acc.at[pl.ds(row0, DEG_PER_TILE)])
    plsc.subcore_barrier()

    half = C // 2

    def body(i, carry):
        cc = c * half + i
        pltpu.sync_copy(ones_v, acc.at[idx_all.at[cc]], add=True)
        return carry

    lax.fori_loop(0, half, body, 0)
    plsc.subcore_barrier()
    pltpu.sync_copy(acc.at[pl.ds(row0, DEG_PER_TILE)], zbuf)
    pltpu.sync_copy(zbuf, deg_out.at[c, pl.ds(row0, DEG_PER_TILE)])


@functools.partial(
    pl.kernel,
    out_type=jax.ShapeDtypeStruct((NC, ACC_ROWS, DH), jnp.float32),
    mesh=_MESH,
    scratch_types=[
        pltpu.VMEM((8, 2, K), jnp.int32),
        [pltpu.VMEM((K, DH), jnp.float32) for _ in range(8)],
        pltpu.VMEM((128, DH), jnp.float32),
        pltpu.VMEM_SHARED((ACC_ROWS, DH), jnp.float32),
        [pltpu.SemaphoreType.DMA for _ in range(8)],
        [pltpu.SemaphoreType.DMA for _ in range(8)],
    ],
    compiler_params=pltpu.CompilerParams(use_tc_tiling_on_sc=False),
)
def _msg_kernel(h2s_hbm, sd4_hbm, zmsg_hbm, part_out,
                sdx, rows, zbuf, acc, sem_g, sem_s):
    c = lax.axis_index("c")
    s = lax.axis_index("s")
    pltpu.sync_copy(zmsg_hbm, zbuf)
    base_row = s * ROWS_PER_TILE
    for off, cnt in _MSG_CHUNKS:
        pltpu.sync_copy(zbuf.at[pl.ds(0, cnt)], acc.at[pl.ds(base_row + off, cnt)])
    plsc.subcore_barrier()

    table = h2s_hbm.at[c]
    NB = 8

    def body(tb, carry):
        for b in range(NB):
            @pl.when(tb > 0)
            def _():
                pltpu.make_async_copy(rows[b], acc.at[sdx.at[b, 1]], sem_s[b]).wait()
        pltpu.sync_copy(sd4_hbm.at[s, pl.ds(tb * NB, NB)], sdx)
        for b in range(NB):
            pltpu.async_copy(table.at[sdx.at[b, 0]], rows[b], sem_g[b])
        for b in range(NB):
            pltpu.make_async_copy(table.at[sdx.at[b, 0]], rows[b], sem_g[b]).wait()
            pltpu.async_copy(rows[b], acc.at[sdx.at[b, 1]], sem_s[b], add=True)
        return carry

    lax.fori_loop(0, C // NB, body, 0)
    for b in range(NB):
        pltpu.make_async_copy(rows[b], acc.at[sdx.at[b, 1]], sem_s[b]).wait()
    plsc.subcore_barrier()
    for off, cnt in _MSG_CHUNKS:
        pltpu.sync_copy(acc.at[pl.ds(base_row + off, cnt)], zbuf.at[pl.ds(0, cnt)])
        pltpu.sync_copy(zbuf.at[pl.ds(0, cnt)], part_out.at[c, pl.ds(base_row + off, cnt)])


R = 1000


def _pre_body(x_ref, d0_ref, d1_ref, w_ref, wr_ref, b_ref, br_ref,
              h2lo_ref, h2hi_ref, base_ref, dis_ref):
    x = x_ref[...]
    deg = 1.0 + d0_ref[...][:, 0:1] + d1_ref[...][:, 0:1]
    dis = lax.rsqrt(deg)
    h2 = jnp.dot(x, w_ref[...], preferred_element_type=jnp.float32) * dis
    base = (h2 * dis
            + jnp.dot(x, wr_ref[...], preferred_element_type=jnp.float32)
            + b_ref[...] + br_ref[...])
    h2lo_ref[...] = h2[:, :DH]
    h2hi_ref[...] = h2[:, DH:]
    base_ref[...] = base
    dis_ref[...] = dis


_pre_call = pl.pallas_call(
    _pre_body,
    grid=(N // R,),
    in_specs=[
        pl.BlockSpec((R, D), lambda i: (i, 0)),
        pl.BlockSpec((R, DEG_W), lambda i: (i, 0)),
        pl.BlockSpec((R, DEG_W), lambda i: (i, 0)),
        pl.BlockSpec((D, D), lambda i: (0, 0)),
        pl.BlockSpec((D, D), lambda i: (0, 0)),
        pl.BlockSpec((1, D), lambda i: (0, 0)),
        pl.BlockSpec((1, D), lambda i: (0, 0)),
    ],
    out_specs=[
        pl.BlockSpec((R, DH), lambda i: (i, 0)),
        pl.BlockSpec((R, DH), lambda i: (i, 0)),
        pl.BlockSpec((R, D), lambda i: (i, 0)),
        pl.BlockSpec((R, 1), lambda i: (i, 0)),
    ],
    out_shape=[
        jax.ShapeDtypeStruct((N, DH), jnp.float32),
        jax.ShapeDtypeStruct((N, DH), jnp.float32),
        jax.ShapeDtypeStruct((N, D), jnp.float32),
        jax.ShapeDtypeStruct((N, 1), jnp.float32),
    ],
)


def _post_body(p_ref, base_ref, dis_ref, o_ref):
    p = p_ref[...]
    agg = jnp.concatenate([p[0], p[1]], axis=1)
    o_ref[...] = jnp.maximum(agg * dis_ref[...] + base_ref[...], 0.0)


_post_call = pl.pallas_call(
    _post_body,
    grid=(N // R,),
    in_specs=[
        pl.BlockSpec((NC, R, DH), lambda i: (0, i, 0)),
        pl.BlockSpec((R, D), lambda i: (i, 0)),
        pl.BlockSpec((R, 1), lambda i: (i, 0)),
    ],
    out_specs=pl.BlockSpec((R, D), lambda i: (i, 0)),
    out_shape=jax.ShapeDtypeStruct((N, D), jnp.float32),
)


def kernel(x, edge_index, W, b, W_res, b_res):
    ei = edge_index.astype(jnp.int32)
    npad = EPAD - E
    pad_src = jnp.zeros((npad,), jnp.int32)
    pad_dst = N + (jnp.arange(npad, dtype=jnp.int32) & 15)
    src3 = jnp.concatenate([ei[0], pad_src]).reshape(NS, C, K)
    dst3 = jnp.concatenate([ei[1], pad_dst]).reshape(NS, C, K)
    sd4 = jnp.stack([src3, dst3], axis=2)

    ones8 = jnp.zeros((K, DEG_W), jnp.float32).at[:, 0].set(1.0)
    zdeg = jnp.zeros((DEG_PER_TILE, DEG_W), jnp.float32)
    zmsg = jnp.zeros((128, DH), jnp.float32)

    deg = _deg_kernel(dst3, ones8, zdeg)
    h2lo, h2hi, base, dis = _pre_call(
        x, deg[0], deg[1], W, W_res, b.reshape(1, D), b_res.reshape(1, D)
    )
    h2s = jnp.stack([h2lo, h2hi], axis=0)
    part = _msg_kernel(h2s, sd4, zmsg)
    return _post_call(part, base, dis)
